# baseline (speedup 1.0000x reference)
import functools

import jax
import jax.numpy as jnp
from jax import lax
from jax.experimental import pallas as pl
from jax.experimental.pallas import tpu as pltpu
from jax.experimental.pallas import tpu_sc as plsc

N = 10000
NPAD = 10240
E = 160000
EPAD = 163840
F_IN = 256
HID = 256
CODE = 128
NC = 2
NS = 16
BT = 512
BD = 512

_MESH = plsc.VectorSubcoreMesh(
    core_axis_name="c", subcore_axis_name="s", num_cores=NC, num_subcores=NS)


def _deg_body(dst2d, ones_hbm, zinit, out, dstb, ones_v, accs):
    c = lax.axis_index("c")
    s = lax.axis_index("s")
    base = (c * NS + s) * 40
    pltpu.sync_copy(dst2d.at[pl.ds(base, 40)], dstb)
    pltpu.sync_copy(ones_hbm, ones_v)
    pltpu.sync_copy(zinit.at[pl.ds(s * 640, 640)],
                    accs.at[pl.ds(s * 640, 640)])
    plsc.subcore_barrier()

    def body(j, carry):
        pltpu.sync_copy(ones_v, accs.at[dstb.at[j]], add=True)
        return carry

    lax.fori_loop(0, 40, body, 0)
    plsc.subcore_barrier()
    pltpu.sync_copy(accs.at[pl.ds(s * 640, 640)],
                    out.at[pl.ds(c * NPAD + s * 640, 640)])


def _sc_degree(dst2d, ones_hbm, zinit):
    k = pl.kernel(
        _deg_body,
        out_type=jax.ShapeDtypeStruct((2 * NPAD, 128), jnp.float32),
        mesh=_MESH,
        scratch_types=[
            pltpu.VMEM((40, 128), jnp.int32),
            pltpu.VMEM((128, 128), jnp.float32),
            pltpu.VMEM_SHARED((NPAD, 128), jnp.float32),
        ],
    )
    return k(dst2d, ones_hbm, zinit)


SS = 40


def _prop_body(colsplit, nch, table, src2d, dst2d, zinit, out,
               srcb, dstb, rows0, rows1, accs, sem0, sem1):
    c = lax.axis_index("c")
    s = lax.axis_index("s")
    if colsplit:
        base = s * nch
    else:
        base = (c * NS + s) * nch

    pltpu.sync_copy(zinit.at[pl.ds(s * 640, 640)], accs.at[pl.ds(s * 640, 640)])
    plsc.subcore_barrier()

    for stage in range(nch // SS):
        sb = base + stage * SS
        pltpu.sync_copy(src2d.at[pl.ds(sb, SS)], srcb)
        pltpu.sync_copy(dst2d.at[pl.ds(sb, SS)], dstb)
        off = c * NPAD

        def addoff(j, carry):
            for k in range(8):
                srcb[j, pl.ds(16 * k, 16)] = srcb[j, pl.ds(16 * k, 16)] + off
            return carry

        lax.fori_loop(0, SS, addoff, 0)

        pltpu.async_copy(table.at[srcb.at[0]], rows0, sem0)
        pltpu.async_copy(table.at[srcb.at[1]], rows1, sem1)

        def pair(i, carry):
            j0 = 2 * i
            pltpu.make_async_copy(table.at[srcb.at[j0]], rows0, sem0).wait()
            pltpu.sync_copy(rows0, accs.at[dstb.at[j0]], add=True)
            pltpu.async_copy(table.at[srcb.at[j0 + 2]], rows0, sem0)
            pltpu.make_async_copy(table.at[srcb.at[j0 + 1]], rows1, sem1).wait()
            pltpu.sync_copy(rows1, accs.at[dstb.at[j0 + 1]], add=True)
            pltpu.async_copy(table.at[srcb.at[j0 + 3]], rows1, sem1)
            return carry

        lax.fori_loop(0, SS // 2 - 1, pair, 0)
        pltpu.make_async_copy(table.at[srcb.at[SS - 2]], rows0, sem0).wait()
        pltpu.sync_copy(rows0, accs.at[dstb.at[SS - 2]], add=True)
        pltpu.make_async_copy(table.at[srcb.at[SS - 1]], rows1, sem1).wait()
        pltpu.sync_copy(rows1, accs.at[dstb.at[SS - 1]], add=True)

    plsc.subcore_barrier()
    pltpu.sync_copy(accs.at[pl.ds(s * 640, 640)],
                    out.at[pl.ds(c * NPAD + s * 640, 640)])


def _sc_propagate(table, src2d, dst2d, zinit, colsplit):
    nch = (EPAD // 128) // NS if colsplit else (EPAD // 128) // (NC * NS)
    k = pl.kernel(
        functools.partial(_prop_body, colsplit, nch),
        out_type=jax.ShapeDtypeStruct((2 * NPAD, 128), jnp.float32),
        mesh=_MESH,
        scratch_types=[
            pltpu.VMEM((SS, 128), jnp.int32),
            pltpu.VMEM((SS, 128), jnp.int32),
            pltpu.VMEM((128, 128), jnp.float32),
            pltpu.VMEM((128, 128), jnp.float32),
            pltpu.VMEM_SHARED((NPAD, 128), jnp.float32),
            pltpu.SemaphoreType.DMA,
            pltpu.SemaphoreType.DMA,
        ],
    )
    return k(table, src2d, dst2d, zinit)


def _xw1_body(x_ref, w_ref, xw_ref):
    xw = jnp.dot(x_ref[...], w_ref[...], preferred_element_type=jnp.float32)
    xw_ref[0] = xw[:, :128]
    xw_ref[1] = xw[:, 128:]


def _tc_xw1(x, W1):
    return pl.pallas_call(
        _xw1_body,
        grid=(NPAD // BT,),
        in_specs=[
            pl.BlockSpec((BT, F_IN), lambda i: (i, 0)),
            pl.BlockSpec((F_IN, HID), lambda i: (0, 0)),
        ],
        out_specs=pl.BlockSpec((2, BT, 128), lambda i: (0, i, 0)),
        out_shape=jax.ShapeDtypeStruct((2, NPAD, 128), jnp.float32),
    )(x, W1)


def _scale1_body(xw_ref, deg_ref, p_ref, dinv_ref):
    d = deg_ref[0] + deg_ref[1] + 1.0
    dcol = lax.rsqrt(d[:, 0:1])
    p_ref[0] = dcol * xw_ref[0]
    p_ref[1] = dcol * xw_ref[1]
    dinv_ref[...] = jnp.broadcast_to(dcol, (BT, 128))


def _tc_scale1(XW, deg2):
    return pl.pallas_call(
        _scale1_body,
        grid=(NPAD // BT,),
        in_specs=[
            pl.BlockSpec((2, BT, 128), lambda i: (0, i, 0)),
            pl.BlockSpec((2, BT, 128), lambda i: (0, i, 0)),
        ],
        out_specs=[
            pl.BlockSpec((2, BT, 128), lambda i: (0, i, 0)),
            pl.BlockSpec((BT, 128), lambda i: (i, 0)),
        ],
        out_shape=[
            jax.ShapeDtypeStruct((2, NPAD, 128), jnp.float32),
            jax.ShapeDtypeStruct((NPAD, 128), jnp.float32),
        ],
    )(XW, deg2)


def _encode2_body(s_ref, p_ref, dinv_ref, w_ref, q_ref):
    din = dinv_ref[...]
    h0 = jnp.maximum(din * (s_ref[0] + p_ref[0]), 0.0)
    h1 = jnp.maximum(din * (s_ref[1] + p_ref[1]), 0.0)
    h = jnp.concatenate([h0, h1], axis=1)
    q = din * jnp.dot(h, w_ref[...], preferred_element_type=jnp.float32)
    q_ref[0] = q
    q_ref[1] = q


def _tc_encode2(S1, P, dinvb, W2):
    return pl.pallas_call(
        _encode2_body,
        grid=(NPAD // BT,),
        in_specs=[
            pl.BlockSpec((2, BT, 128), lambda i: (0, i, 0)),
            pl.BlockSpec((2, BT, 128), lambda i: (0, i, 0)),
            pl.BlockSpec((BT, 128), lambda i: (i, 0)),
            pl.BlockSpec((HID, CODE), lambda i: (0, 0)),
        ],
        out_specs=pl.BlockSpec((2, BT, 128), lambda i: (0, i, 0)),
        out_shape=jax.ShapeDtypeStruct((2, NPAD, 128), jnp.float32),
    )(S1, P, dinvb, W2)


def _decode_body(si_ref, qi_ref, di_ref, sj_ref, qj_ref, dj_ref, o_ref):
    zi = di_ref[...] * (si_ref[0] + si_ref[1] + qi_ref[0])
    zj = dj_ref[...] * (sj_ref[0] + sj_ref[1] + qj_ref[0])
    zz = lax.dot_general(zi, zj, (((1,), (1,)), ((), ())),
                         preferred_element_type=jnp.float32)
    o_ref[...] = jax.nn.sigmoid(zz)


def _tc_decode(S2, Q2, dinvb):
    return pl.pallas_call(
        _decode_body,
        grid=(pl.cdiv(N, BD), pl.cdiv(N, BD)),
        in_specs=[
            pl.BlockSpec((2, BD, CODE), lambda i, j: (0, i, 0)),
            pl.BlockSpec((2, BD, CODE), lambda i, j: (0, i, 0)),
            pl.BlockSpec((BD, CODE), lambda i, j: (i, 0)),
            pl.BlockSpec((2, BD, CODE), lambda i, j: (0, j, 0)),
            pl.BlockSpec((2, BD, CODE), lambda i, j: (0, j, 0)),
            pl.BlockSpec((BD, CODE), lambda i, j: (j, 0)),
        ],
        out_specs=pl.BlockSpec((BD, BD), lambda i, j: (i, j)),
        out_shape=jax.ShapeDtypeStruct((N, N), jnp.float32),
    )(S2, Q2, dinvb, S2, Q2, dinvb)


def kernel(x, edge_index, W1, W2):
    src = edge_index[0].astype(jnp.int32)
    dst = edge_index[1].astype(jnp.int32)
    src2d = jnp.concatenate(
        [src, jnp.zeros((EPAD - E,), jnp.int32)]).reshape(EPAD // 128, 128)
    pad_dst = N + jnp.arange(EPAD - E, dtype=jnp.int32) % (NPAD - N)
    dst2d = jnp.concatenate([dst, pad_dst]).reshape(EPAD // 128, 128)
    zinit = jnp.zeros((NPAD, 128), jnp.float32)
    ones128 = jnp.ones((128, 128), jnp.float32)
    xp = jnp.pad(x, ((0, NPAD - N), (0, 0)))

    deg2 = _sc_degree(dst2d, ones128, zinit).reshape(2, NPAD, 128)
    XW = _tc_xw1(xp, W1)
    P, dinvb = _tc_scale1(XW, deg2)
    S1 = _sc_propagate(P.reshape(2 * NPAD, 128), src2d, dst2d, zinit,
                       colsplit=True).reshape(2, NPAD, 128)
    Q2 = _tc_encode2(S1, P, dinvb, W2)
    S2 = _sc_propagate(Q2.reshape(2 * NPAD, 128), src2d, dst2d, zinit,
                       colsplit=False).reshape(2, NPAD, 128)
    return _tc_decode(S2, Q2, dinvb)


# speedup vs baseline: 1.1126x; 1.1126x over previous
import functools

import jax
import jax.numpy as jnp
from jax import lax
from jax.experimental import pallas as pl
from jax.experimental.pallas import tpu as pltpu
from jax.experimental.pallas import tpu_sc as plsc

N = 10000
NPAD = 10240
E = 160000
EPAD = 163840
F_IN = 256
HID = 256
CODE = 128
NC = 2
NS = 16
BT = 512
BD = 512

_MESH = plsc.VectorSubcoreMesh(
    core_axis_name="c", subcore_axis_name="s", num_cores=NC, num_subcores=NS)


def _deg_body(dst2d, ones_hbm, zinit, out, dstb, ones_v, accs):
    c = lax.axis_index("c")
    s = lax.axis_index("s")
    base = (c * NS + s) * 40
    pltpu.sync_copy(dst2d.at[pl.ds(base, 40)], dstb)
    pltpu.sync_copy(ones_hbm, ones_v)
    pltpu.sync_copy(zinit.at[pl.ds(s * 640, 640)],
                    accs.at[pl.ds(s * 640, 640)])
    plsc.subcore_barrier()

    def body(j, carry):
        pltpu.sync_copy(ones_v, accs.at[dstb.at[j]], add=True)
        return carry

    lax.fori_loop(0, 40, body, 0)
    plsc.subcore_barrier()
    pltpu.sync_copy(accs.at[pl.ds(s * 640, 640)],
                    out.at[pl.ds(c * NPAD + s * 640, 640)])


def _sc_degree(dst2d, ones_hbm, zinit):
    k = pl.kernel(
        _deg_body,
        out_type=jax.ShapeDtypeStruct((2 * NPAD, 128), jnp.float32),
        mesh=_MESH,
        scratch_types=[
            pltpu.VMEM((40, 128), jnp.int32),
            pltpu.VMEM((128, 128), jnp.float32),
            pltpu.VMEM_SHARED((NPAD, 128), jnp.float32),
        ],
    )
    return k(dst2d, ones_hbm, zinit)


SS = 40


def _prop_body(colsplit, nch, table, src2d, dst2d, zinit, out,
               srcb, dstb, rows0, rows1, accs, sem0, sem1):
    c = lax.axis_index("c")
    s = lax.axis_index("s")
    if colsplit:
        base = s * nch
    else:
        base = (c * NS + s) * nch

    pltpu.sync_copy(zinit.at[pl.ds(s * 640, 640)], accs.at[pl.ds(s * 640, 640)])
    plsc.subcore_barrier()

    for stage in range(nch // SS):
        sb = base + stage * SS
        pltpu.sync_copy(src2d.at[pl.ds(sb, SS)], srcb)
        pltpu.sync_copy(dst2d.at[pl.ds(sb, SS)], dstb)
        off = c * NPAD

        def addoff(j, carry):
            for k in range(8):
                srcb[j, pl.ds(16 * k, 16)] = srcb[j, pl.ds(16 * k, 16)] + off
            return carry

        lax.fori_loop(0, SS, addoff, 0)

        pltpu.async_copy(table.at[srcb.at[0]], rows0, sem0)
        pltpu.async_copy(table.at[srcb.at[1]], rows1, sem1)

        def pair(i, carry):
            j0 = 2 * i
            pltpu.make_async_copy(table.at[srcb.at[j0]], rows0, sem0).wait()
            pltpu.sync_copy(rows0, accs.at[dstb.at[j0]], add=True)
            pltpu.async_copy(table.at[srcb.at[j0 + 2]], rows0, sem0)
            pltpu.make_async_copy(table.at[srcb.at[j0 + 1]], rows1, sem1).wait()
            pltpu.sync_copy(rows1, accs.at[dstb.at[j0 + 1]], add=True)
            pltpu.async_copy(table.at[srcb.at[j0 + 3]], rows1, sem1)
            return carry

        lax.fori_loop(0, SS // 2 - 1, pair, 0)
        pltpu.make_async_copy(table.at[srcb.at[SS - 2]], rows0, sem0).wait()
        pltpu.sync_copy(rows0, accs.at[dstb.at[SS - 2]], add=True)
        pltpu.make_async_copy(table.at[srcb.at[SS - 1]], rows1, sem1).wait()
        pltpu.sync_copy(rows1, accs.at[dstb.at[SS - 1]], add=True)

    plsc.subcore_barrier()
    pltpu.sync_copy(accs.at[pl.ds(s * 640, 640)],
                    out.at[pl.ds(c * NPAD + s * 640, 640)])


def _sc_propagate(table, src2d, dst2d, zinit, colsplit):
    nch = (EPAD // 128) // NS if colsplit else (EPAD // 128) // (NC * NS)
    k = pl.kernel(
        functools.partial(_prop_body, colsplit, nch),
        out_type=jax.ShapeDtypeStruct((2 * NPAD, 128), jnp.float32),
        mesh=_MESH,
        scratch_types=[
            pltpu.VMEM((SS, 128), jnp.int32),
            pltpu.VMEM((SS, 128), jnp.int32),
            pltpu.VMEM((128, 128), jnp.float32),
            pltpu.VMEM((128, 128), jnp.float32),
            pltpu.VMEM_SHARED((NPAD, 128), jnp.float32),
            pltpu.SemaphoreType.DMA,
            pltpu.SemaphoreType.DMA,
        ],
    )
    return k(table, src2d, dst2d, zinit)


def _xw1_body(x_ref, w_ref, xw_ref):
    xw = jnp.dot(x_ref[...], w_ref[...], preferred_element_type=jnp.float32)
    xw_ref[0] = xw[:, :128]
    xw_ref[1] = xw[:, 128:]


def _tc_xw1(x, W1):
    return pl.pallas_call(
        _xw1_body,
        grid=(NPAD // BT,),
        in_specs=[
            pl.BlockSpec((BT, F_IN), lambda i: (i, 0)),
            pl.BlockSpec((F_IN, HID), lambda i: (0, 0)),
        ],
        out_specs=pl.BlockSpec((2, BT, 128), lambda i: (0, i, 0)),
        out_shape=jax.ShapeDtypeStruct((2, NPAD, 128), jnp.float32),
    )(x, W1)


def _scale1_body(xw_ref, deg_ref, p_ref, dinv_ref):
    d = deg_ref[0] + deg_ref[1] + 1.0
    dcol = lax.rsqrt(d[:, 0:1])
    p_ref[0] = dcol * xw_ref[0]
    p_ref[1] = dcol * xw_ref[1]
    dinv_ref[...] = jnp.broadcast_to(dcol, (BT, 128))


def _tc_scale1(XW, deg2):
    return pl.pallas_call(
        _scale1_body,
        grid=(NPAD // BT,),
        in_specs=[
            pl.BlockSpec((2, BT, 128), lambda i: (0, i, 0)),
            pl.BlockSpec((2, BT, 128), lambda i: (0, i, 0)),
        ],
        out_specs=[
            pl.BlockSpec((2, BT, 128), lambda i: (0, i, 0)),
            pl.BlockSpec((BT, 128), lambda i: (i, 0)),
        ],
        out_shape=[
            jax.ShapeDtypeStruct((2, NPAD, 128), jnp.float32),
            jax.ShapeDtypeStruct((NPAD, 128), jnp.float32),
        ],
    )(XW, deg2)


def _encode2_body(s_ref, p_ref, dinv_ref, w_ref, q_ref):
    din = dinv_ref[...]
    h0 = jnp.maximum(din * (s_ref[0] + p_ref[0]), 0.0)
    h1 = jnp.maximum(din * (s_ref[1] + p_ref[1]), 0.0)
    h = jnp.concatenate([h0, h1], axis=1)
    q = din * jnp.dot(h, w_ref[...], preferred_element_type=jnp.float32)
    q_ref[0] = q
    q_ref[1] = q


def _tc_encode2(S1, P, dinvb, W2):
    return pl.pallas_call(
        _encode2_body,
        grid=(NPAD // BT,),
        in_specs=[
            pl.BlockSpec((2, BT, 128), lambda i: (0, i, 0)),
            pl.BlockSpec((2, BT, 128), lambda i: (0, i, 0)),
            pl.BlockSpec((BT, 128), lambda i: (i, 0)),
            pl.BlockSpec((HID, CODE), lambda i: (0, 0)),
        ],
        out_specs=pl.BlockSpec((2, BT, 128), lambda i: (0, i, 0)),
        out_shape=jax.ShapeDtypeStruct((2, NPAD, 128), jnp.float32),
    )(S1, P, dinvb, W2)


def _z_body(s_ref, q_ref, dinv_ref, z_ref):
    z_ref[...] = dinv_ref[...] * (s_ref[0] + s_ref[1] + q_ref[0])


def _tc_z(S2, Q2, dinvb):
    return pl.pallas_call(
        _z_body,
        grid=(NPAD // BT,),
        in_specs=[
            pl.BlockSpec((2, BT, 128), lambda i: (0, i, 0)),
            pl.BlockSpec((2, BT, 128), lambda i: (0, i, 0)),
            pl.BlockSpec((BT, 128), lambda i: (i, 0)),
        ],
        out_specs=pl.BlockSpec((BT, 128), lambda i: (i, 0)),
        out_shape=jax.ShapeDtypeStruct((NPAD, 128), jnp.float32),
    )(S2, Q2, dinvb)


def _decode_body(zi_ref, zj_ref, o_ref):
    zz = lax.dot_general(zi_ref[...], zj_ref[...],
                         (((1,), (1,)), ((), ())),
                         preferred_element_type=jnp.float32)
    o_ref[...] = jax.nn.sigmoid(zz)


def _tc_decode(z):
    return pl.pallas_call(
        _decode_body,
        grid=(pl.cdiv(N, BD), pl.cdiv(N, BD)),
        in_specs=[
            pl.BlockSpec((BD, CODE), lambda i, j: (i, 0)),
            pl.BlockSpec((BD, CODE), lambda i, j: (j, 0)),
        ],
        out_specs=pl.BlockSpec((BD, BD), lambda i, j: (i, j)),
        out_shape=jax.ShapeDtypeStruct((N, N), jnp.float32),
    )(z, z)


def kernel(x, edge_index, W1, W2):
    src = edge_index[0].astype(jnp.int32)
    dst = edge_index[1].astype(jnp.int32)
    src2d = jnp.concatenate(
        [src, jnp.zeros((EPAD - E,), jnp.int32)]).reshape(EPAD // 128, 128)
    pad_dst = N + jnp.arange(EPAD - E, dtype=jnp.int32) % (NPAD - N)
    dst2d = jnp.concatenate([dst, pad_dst]).reshape(EPAD // 128, 128)
    zinit = jnp.zeros((NPAD, 128), jnp.float32)
    ones128 = jnp.ones((128, 128), jnp.float32)
    xp = jnp.pad(x, ((0, NPAD - N), (0, 0)))

    deg2 = _sc_degree(dst2d, ones128, zinit).reshape(2, NPAD, 128)
    XW = _tc_xw1(xp, W1)
    P, dinvb = _tc_scale1(XW, deg2)
    S1 = _sc_propagate(P.reshape(2 * NPAD, 128), src2d, dst2d, zinit,
                       colsplit=True).reshape(2, NPAD, 128)
    Q2 = _tc_encode2(S1, P, dinvb, W2)
    S2 = _sc_propagate(Q2.reshape(2 * NPAD, 128), src2d, dst2d, zinit,
                       colsplit=False).reshape(2, NPAD, 128)
    z = _tc_z(S2, Q2, dinvb)
    return _tc_decode(z)


# speedup vs baseline: 1.1229x; 1.0093x over previous
import functools

import jax
import jax.numpy as jnp
from jax import lax
from jax.experimental import pallas as pl
from jax.experimental.pallas import tpu as pltpu
from jax.experimental.pallas import tpu_sc as plsc

N = 10000
NPAD = 10240
E = 160000
EPAD = 163840
F_IN = 256
HID = 256
CODE = 128
NC = 2
NS = 16
BT = 512
BD = 512

_MESH = plsc.VectorSubcoreMesh(
    core_axis_name="c", subcore_axis_name="s", num_cores=NC, num_subcores=NS)


def _deg_body(dst2d, ones_hbm, zinit, out, dstb, ones_v, accs):
    c = lax.axis_index("c")
    s = lax.axis_index("s")
    base = (c * NS + s) * 40
    pltpu.sync_copy(dst2d.at[pl.ds(base, 40)], dstb)
    pltpu.sync_copy(ones_hbm, ones_v)
    pltpu.sync_copy(zinit.at[pl.ds(s * 640, 640)],
                    accs.at[pl.ds(s * 640, 640)])
    plsc.subcore_barrier()

    def body(j, carry):
        pltpu.sync_copy(ones_v, accs.at[dstb.at[j]], add=True)
        return carry

    lax.fori_loop(0, 40, body, 0)
    plsc.subcore_barrier()
    pltpu.sync_copy(accs.at[pl.ds(s * 640, 640)],
                    out.at[pl.ds(c * NPAD + s * 640, 640)])


def _sc_degree(dst2d, ones_hbm, zinit):
    k = pl.kernel(
        _deg_body,
        out_type=jax.ShapeDtypeStruct((2 * NPAD, 128), jnp.float32),
        mesh=_MESH,
        scratch_types=[
            pltpu.VMEM((40, 128), jnp.int32),
            pltpu.VMEM((128, 128), jnp.float32),
            pltpu.VMEM_SHARED((NPAD, 128), jnp.float32),
        ],
    )
    return k(dst2d, ones_hbm, zinit)


SS = 40


def _prop_body(colsplit, nch, table, src2d, dst2d, zinit, out,
               srcb, dstb, rows0, rows1, accs, sem0, sem1):
    c = lax.axis_index("c")
    s = lax.axis_index("s")
    if colsplit:
        base = s * nch
    else:
        base = (c * NS + s) * nch

    pltpu.sync_copy(zinit.at[pl.ds(s * 640, 640)], accs.at[pl.ds(s * 640, 640)])
    plsc.subcore_barrier()

    for stage in range(nch // SS):
        sb = base + stage * SS
        pltpu.sync_copy(src2d.at[pl.ds(sb, SS)], srcb)
        pltpu.sync_copy(dst2d.at[pl.ds(sb, SS)], dstb)
        off = c * NPAD

        def addoff(j, carry):
            for k in range(8):
                srcb[j, pl.ds(16 * k, 16)] = srcb[j, pl.ds(16 * k, 16)] + off
            return carry

        lax.fori_loop(0, SS, addoff, 0)

        pltpu.async_copy(table.at[srcb.at[0]], rows0, sem0)
        pltpu.async_copy(table.at[srcb.at[1]], rows1, sem1)

        def pair(i, carry):
            j0 = 2 * i
            pltpu.make_async_copy(table.at[srcb.at[j0]], rows0, sem0).wait()
            pltpu.sync_copy(rows0, accs.at[dstb.at[j0]], add=True)
            pltpu.async_copy(table.at[srcb.at[j0 + 2]], rows0, sem0)
            pltpu.make_async_copy(table.at[srcb.at[j0 + 1]], rows1, sem1).wait()
            pltpu.sync_copy(rows1, accs.at[dstb.at[j0 + 1]], add=True)
            pltpu.async_copy(table.at[srcb.at[j0 + 3]], rows1, sem1)
            return carry

        lax.fori_loop(0, SS // 2 - 1, pair, 0)
        pltpu.make_async_copy(table.at[srcb.at[SS - 2]], rows0, sem0).wait()
        pltpu.sync_copy(rows0, accs.at[dstb.at[SS - 2]], add=True)
        pltpu.make_async_copy(table.at[srcb.at[SS - 1]], rows1, sem1).wait()
        pltpu.sync_copy(rows1, accs.at[dstb.at[SS - 1]], add=True)

    plsc.subcore_barrier()
    pltpu.sync_copy(accs.at[pl.ds(s * 640, 640)],
                    out.at[pl.ds(c * NPAD + s * 640, 640)])


def _sc_propagate(table, src2d, dst2d, zinit, colsplit):
    nch = (EPAD // 128) // NS if colsplit else (EPAD // 128) // (NC * NS)
    k = pl.kernel(
        functools.partial(_prop_body, colsplit, nch),
        out_type=jax.ShapeDtypeStruct((2 * NPAD, 128), jnp.float32),
        mesh=_MESH,
        scratch_types=[
            pltpu.VMEM((SS, 128), jnp.int32),
            pltpu.VMEM((SS, 128), jnp.int32),
            pltpu.VMEM((128, 128), jnp.float32),
            pltpu.VMEM((128, 128), jnp.float32),
            pltpu.VMEM_SHARED((NPAD, 128), jnp.float32),
            pltpu.SemaphoreType.DMA,
            pltpu.SemaphoreType.DMA,
        ],
    )
    return k(table, src2d, dst2d, zinit)


def _encode1_body(x_ref, w_ref, deg_ref, p_ref, dinv_ref):
    d = deg_ref[0] + deg_ref[1] + 1.0
    dcol = lax.rsqrt(d[:, 0:1])
    xw = jnp.dot(x_ref[...], w_ref[...], preferred_element_type=jnp.float32)
    p = dcol * xw
    p_ref[0] = p[:, :128]
    p_ref[1] = p[:, 128:]
    dinv_ref[...] = jnp.broadcast_to(dcol, (BT, 128))


def _tc_encode1(x, W1, deg2):
    return pl.pallas_call(
        _encode1_body,
        grid=(NPAD // BT,),
        in_specs=[
            pl.BlockSpec((BT, F_IN), lambda i: (i, 0)),
            pl.BlockSpec((F_IN, HID), lambda i: (0, 0)),
            pl.BlockSpec((2, BT, 128), lambda i: (0, i, 0)),
        ],
        out_specs=[
            pl.BlockSpec((2, BT, 128), lambda i: (0, i, 0)),
            pl.BlockSpec((BT, 128), lambda i: (i, 0)),
        ],
        out_shape=[
            jax.ShapeDtypeStruct((2, NPAD, 128), jnp.float32),
            jax.ShapeDtypeStruct((NPAD, 128), jnp.float32),
        ],
    )(x, W1, deg2)


def _encode2_body(s_ref, p_ref, dinv_ref, w_ref, q_ref):
    din = dinv_ref[...]
    h0 = jnp.maximum(din * (s_ref[0] + p_ref[0]), 0.0)
    h1 = jnp.maximum(din * (s_ref[1] + p_ref[1]), 0.0)
    h = jnp.concatenate([h0, h1], axis=1)
    q = din * jnp.dot(h, w_ref[...], preferred_element_type=jnp.float32)
    q_ref[0] = q
    q_ref[1] = q


def _tc_encode2(S1, P, dinvb, W2):
    return pl.pallas_call(
        _encode2_body,
        grid=(NPAD // BT,),
        in_specs=[
            pl.BlockSpec((2, BT, 128), lambda i: (0, i, 0)),
            pl.BlockSpec((2, BT, 128), lambda i: (0, i, 0)),
            pl.BlockSpec((BT, 128), lambda i: (i, 0)),
            pl.BlockSpec((HID, CODE), lambda i: (0, 0)),
        ],
        out_specs=pl.BlockSpec((2, BT, 128), lambda i: (0, i, 0)),
        out_shape=jax.ShapeDtypeStruct((2, NPAD, 128), jnp.float32),
    )(S1, P, dinvb, W2)


def _z_body(s_ref, q_ref, dinv_ref, z_ref):
    z_ref[...] = dinv_ref[...] * (s_ref[0] + s_ref[1] + q_ref[0])


def _tc_z(S2, Q2, dinvb):
    return pl.pallas_call(
        _z_body,
        grid=(NPAD // BT,),
        in_specs=[
            pl.BlockSpec((2, BT, 128), lambda i: (0, i, 0)),
            pl.BlockSpec((2, BT, 128), lambda i: (0, i, 0)),
            pl.BlockSpec((BT, 128), lambda i: (i, 0)),
        ],
        out_specs=pl.BlockSpec((BT, 128), lambda i: (i, 0)),
        out_shape=jax.ShapeDtypeStruct((NPAD, 128), jnp.float32),
    )(S2, Q2, dinvb)


def _decode_body(zi_ref, zj_ref, o_ref):
    zz = lax.dot_general(zi_ref[...], zj_ref[...],
                         (((1,), (1,)), ((), ())),
                         preferred_element_type=jnp.float32)
    o_ref[...] = jax.nn.sigmoid(zz)


def _tc_decode(z):
    return pl.pallas_call(
        _decode_body,
        grid=(pl.cdiv(N, BD), pl.cdiv(N, BD)),
        in_specs=[
            pl.BlockSpec((BD, CODE), lambda i, j: (i, 0)),
            pl.BlockSpec((BD, CODE), lambda i, j: (j, 0)),
        ],
        out_specs=pl.BlockSpec((BD, BD), lambda i, j: (i, j)),
        out_shape=jax.ShapeDtypeStruct((N, N), jnp.float32),
    )(z, z)


def kernel(x, edge_index, W1, W2):
    src = edge_index[0].astype(jnp.int32)
    dst = edge_index[1].astype(jnp.int32)
    src2d = jnp.concatenate(
        [src, jnp.zeros((EPAD - E,), jnp.int32)]).reshape(EPAD // 128, 128)
    pad_dst = N + jnp.arange(EPAD - E, dtype=jnp.int32) % (NPAD - N)
    dst2d = jnp.concatenate([dst, pad_dst]).reshape(EPAD // 128, 128)
    zinit = jnp.zeros((NPAD, 128), jnp.float32)
    ones128 = jnp.ones((128, 128), jnp.float32)
    xp = jnp.pad(x, ((0, NPAD - N), (0, 0)))

    deg2 = _sc_degree(dst2d, ones128, zinit).reshape(2, NPAD, 128)
    P, dinvb = _tc_encode1(xp, W1, deg2)
    S1 = _sc_propagate(P.reshape(2 * NPAD, 128), src2d, dst2d, zinit,
                       colsplit=True).reshape(2, NPAD, 128)
    Q2 = _tc_encode2(S1, P, dinvb, W2)
    S2 = _sc_propagate(Q2.reshape(2 * NPAD, 128), src2d, dst2d, zinit,
                       colsplit=False).reshape(2, NPAD, 128)
    z = _tc_z(S2, Q2, dinvb)
    return _tc_decode(z)


# speedup vs baseline: 1.3820x; 1.2307x over previous
import functools

import jax
import jax.numpy as jnp
from jax import lax
from jax.experimental import pallas as pl
from jax.experimental.pallas import tpu as pltpu
from jax.experimental.pallas import tpu_sc as plsc

N = 10000
NPAD = 10240
E = 160000
EPAD = 163840
F_IN = 256
HID = 256
CODE = 128
NC = 2
NS = 16
BT = 512
BD = 1024

_MESH = plsc.VectorSubcoreMesh(
    core_axis_name="c", subcore_axis_name="s", num_cores=NC, num_subcores=NS)


def _deg_body(dst2d, ones_hbm, zinit, out, dstb, ones_v, accs):
    c = lax.axis_index("c")
    s = lax.axis_index("s")
    base = (c * NS + s) * 40
    pltpu.sync_copy(dst2d.at[pl.ds(base, 40)], dstb)
    pltpu.sync_copy(ones_hbm, ones_v)
    pltpu.sync_copy(zinit.at[pl.ds(s * 640, 640)],
                    accs.at[pl.ds(s * 640, 640)])
    plsc.subcore_barrier()

    def body(j, carry):
        pltpu.sync_copy(ones_v, accs.at[dstb.at[j]], add=True)
        return carry

    lax.fori_loop(0, 40, body, 0)
    plsc.subcore_barrier()
    pltpu.sync_copy(accs.at[pl.ds(s * 640, 640)],
                    out.at[pl.ds(c * NPAD + s * 640, 640)])


def _sc_degree(dst2d, ones_hbm, zinit):
    k = pl.kernel(
        _deg_body,
        out_type=jax.ShapeDtypeStruct((2 * NPAD, 128), jnp.float32),
        mesh=_MESH,
        scratch_types=[
            pltpu.VMEM((40, 128), jnp.int32),
            pltpu.VMEM((128, 128), jnp.float32),
            pltpu.VMEM_SHARED((NPAD, 128), jnp.float32),
        ],
    )
    return k(dst2d, ones_hbm, zinit)


SS = 40


def _prop_body(colsplit, nch, table, src2d, dst2d, zinit, out,
               srcb, dstb, rows0, rows1, accs, sem0, sem1):
    c = lax.axis_index("c")
    s = lax.axis_index("s")
    if colsplit:
        base = s * nch
    else:
        base = (c * NS + s) * nch

    pltpu.sync_copy(zinit.at[pl.ds(s * 640, 640)], accs.at[pl.ds(s * 640, 640)])
    plsc.subcore_barrier()

    for stage in range(nch // SS):
        sb = base + stage * SS
        pltpu.sync_copy(src2d.at[pl.ds(sb, SS)], srcb)
        pltpu.sync_copy(dst2d.at[pl.ds(sb, SS)], dstb)
        off = c * NPAD

        def addoff(j, carry):
            for k in range(8):
                srcb[j, pl.ds(16 * k, 16)] = srcb[j, pl.ds(16 * k, 16)] + off
            return carry

        lax.fori_loop(0, SS, addoff, 0)

        pltpu.async_copy(table.at[srcb.at[0]], rows0, sem0)
        pltpu.async_copy(table.at[srcb.at[1]], rows1, sem1)

        def pair(i, carry):
            j0 = 2 * i
            pltpu.make_async_copy(table.at[srcb.at[j0]], rows0, sem0).wait()
            pltpu.sync_copy(rows0, accs.at[dstb.at[j0]], add=True)
            pltpu.async_copy(table.at[srcb.at[j0 + 2]], rows0, sem0)
            pltpu.make_async_copy(table.at[srcb.at[j0 + 1]], rows1, sem1).wait()
            pltpu.sync_copy(rows1, accs.at[dstb.at[j0 + 1]], add=True)
            pltpu.async_copy(table.at[srcb.at[j0 + 3]], rows1, sem1)
            return carry

        lax.fori_loop(0, SS // 2 - 1, pair, 0)
        pltpu.make_async_copy(table.at[srcb.at[SS - 2]], rows0, sem0).wait()
        pltpu.sync_copy(rows0, accs.at[dstb.at[SS - 2]], add=True)
        pltpu.make_async_copy(table.at[srcb.at[SS - 1]], rows1, sem1).wait()
        pltpu.sync_copy(rows1, accs.at[dstb.at[SS - 1]], add=True)

    plsc.subcore_barrier()
    pltpu.sync_copy(accs.at[pl.ds(s * 640, 640)],
                    out.at[pl.ds(c * NPAD + s * 640, 640)])


def _sc_propagate(table, src2d, dst2d, zinit, colsplit):
    nch = (EPAD // 128) // NS if colsplit else (EPAD // 128) // (NC * NS)
    k = pl.kernel(
        functools.partial(_prop_body, colsplit, nch),
        out_type=jax.ShapeDtypeStruct((2 * NPAD, 128), jnp.float32),
        mesh=_MESH,
        scratch_types=[
            pltpu.VMEM((SS, 128), jnp.int32),
            pltpu.VMEM((SS, 128), jnp.int32),
            pltpu.VMEM((128, 128), jnp.float32),
            pltpu.VMEM((128, 128), jnp.float32),
            pltpu.VMEM_SHARED((NPAD, 128), jnp.float32),
            pltpu.SemaphoreType.DMA,
            pltpu.SemaphoreType.DMA,
        ],
    )
    return k(table, src2d, dst2d, zinit)


def _encode1_body(x_ref, w_ref, deg_ref, p_ref, dinv_ref):
    d = deg_ref[0] + deg_ref[1] + 1.0
    dcol = lax.rsqrt(d[:, 0:1])
    xw = jnp.dot(x_ref[...], w_ref[...], preferred_element_type=jnp.float32)
    p = dcol * xw
    p_ref[0] = p[:, :128]
    p_ref[1] = p[:, 128:]
    dinv_ref[...] = jnp.broadcast_to(dcol, (BT, 128))


def _tc_encode1(x, W1, deg2):
    return pl.pallas_call(
        _encode1_body,
        grid=(NPAD // BT,),
        in_specs=[
            pl.BlockSpec((BT, F_IN), lambda i: (i, 0)),
            pl.BlockSpec((F_IN, HID), lambda i: (0, 0)),
            pl.BlockSpec((2, BT, 128), lambda i: (0, i, 0)),
        ],
        out_specs=[
            pl.BlockSpec((2, BT, 128), lambda i: (0, i, 0)),
            pl.BlockSpec((BT, 128), lambda i: (i, 0)),
        ],
        out_shape=[
            jax.ShapeDtypeStruct((2, NPAD, 128), jnp.float32),
            jax.ShapeDtypeStruct((NPAD, 128), jnp.float32),
        ],
    )(x, W1, deg2)


def _encode2_body(s_ref, p_ref, dinv_ref, w_ref, q_ref):
    din = dinv_ref[...]
    h0 = jnp.maximum(din * (s_ref[0] + p_ref[0]), 0.0)
    h1 = jnp.maximum(din * (s_ref[1] + p_ref[1]), 0.0)
    h = jnp.concatenate([h0, h1], axis=1)
    q = din * jnp.dot(h, w_ref[...], preferred_element_type=jnp.float32)
    q_ref[0] = q
    q_ref[1] = q


def _tc_encode2(S1, P, dinvb, W2):
    return pl.pallas_call(
        _encode2_body,
        grid=(NPAD // BT,),
        in_specs=[
            pl.BlockSpec((2, BT, 128), lambda i: (0, i, 0)),
            pl.BlockSpec((2, BT, 128), lambda i: (0, i, 0)),
            pl.BlockSpec((BT, 128), lambda i: (i, 0)),
            pl.BlockSpec((HID, CODE), lambda i: (0, 0)),
        ],
        out_specs=pl.BlockSpec((2, BT, 128), lambda i: (0, i, 0)),
        out_shape=jax.ShapeDtypeStruct((2, NPAD, 128), jnp.float32),
    )(S1, P, dinvb, W2)


def _z_body(s_ref, q_ref, dinv_ref, z_ref):
    z_ref[...] = dinv_ref[...] * (s_ref[0] + s_ref[1] + q_ref[0])


def _tc_z(S2, Q2, dinvb):
    return pl.pallas_call(
        _z_body,
        grid=(NPAD // BT,),
        in_specs=[
            pl.BlockSpec((2, BT, 128), lambda i: (0, i, 0)),
            pl.BlockSpec((2, BT, 128), lambda i: (0, i, 0)),
            pl.BlockSpec((BT, 128), lambda i: (i, 0)),
        ],
        out_specs=pl.BlockSpec((BT, 128), lambda i: (i, 0)),
        out_shape=jax.ShapeDtypeStruct((NPAD, 128), jnp.float32),
    )(S2, Q2, dinvb)


def _decode_body(zi_ref, zj_ref, o_ref):
    zz = lax.dot_general(zi_ref[...], zj_ref[...],
                         (((1,), (1,)), ((), ())),
                         preferred_element_type=jnp.float32)
    o_ref[...] = jax.nn.sigmoid(zz)


def _tc_decode(z):
    return pl.pallas_call(
        _decode_body,
        grid=(pl.cdiv(N, BD), pl.cdiv(N, BD)),
        in_specs=[
            pl.BlockSpec((BD, CODE), lambda i, j: (i, 0)),
            pl.BlockSpec((BD, CODE), lambda i, j: (j, 0)),
        ],
        out_specs=pl.BlockSpec((BD, BD), lambda i, j: (i, j)),
        out_shape=jax.ShapeDtypeStruct((N, N), jnp.float32),
    )(z, z)


def kernel(x, edge_index, W1, W2):
    src = edge_index[0].astype(jnp.int32)
    dst = edge_index[1].astype(jnp.int32)
    src2d = jnp.concatenate(
        [src, jnp.zeros((EPAD - E,), jnp.int32)]).reshape(EPAD // 128, 128)
    pad_dst = N + jnp.arange(EPAD - E, dtype=jnp.int32) % (NPAD - N)
    dst2d = jnp.concatenate([dst, pad_dst]).reshape(EPAD // 128, 128)
    zinit = jnp.zeros((NPAD, 128), jnp.float32)
    ones128 = jnp.ones((128, 128), jnp.float32)
    xp = jnp.pad(x, ((0, NPAD - N), (0, 0)))

    deg2 = _sc_degree(dst2d, ones128, zinit).reshape(2, NPAD, 128)
    P, dinvb = _tc_encode1(xp, W1, deg2)
    S1 = _sc_propagate(P.reshape(2 * NPAD, 128), src2d, dst2d, zinit,
                       colsplit=True).reshape(2, NPAD, 128)
    Q2 = _tc_encode2(S1, P, dinvb, W2)
    S2 = _sc_propagate(Q2.reshape(2 * NPAD, 128), src2d, dst2d, zinit,
                       colsplit=False).reshape(2, NPAD, 128)
    z = _tc_z(S2, Q2, dinvb)
    return _tc_decode(z)


# speedup vs baseline: 1.4565x; 1.0539x over previous
import functools

import jax
import jax.numpy as jnp
from jax import lax
from jax.experimental import pallas as pl
from jax.experimental.pallas import tpu as pltpu
from jax.experimental.pallas import tpu_sc as plsc

N = 10000
NPAD = 10240
E = 160000
EPAD = 163840
F_IN = 256
HID = 256
CODE = 128
NC = 2
NS = 16
BT = 512
BD = 2048

_MESH = plsc.VectorSubcoreMesh(
    core_axis_name="c", subcore_axis_name="s", num_cores=NC, num_subcores=NS)


def _deg_body(dst2d, ones_hbm, zinit, out, dstb, ones_v, accs):
    c = lax.axis_index("c")
    s = lax.axis_index("s")
    base = (c * NS + s) * 40
    pltpu.sync_copy(dst2d.at[pl.ds(base, 40)], dstb)
    pltpu.sync_copy(ones_hbm, ones_v)
    pltpu.sync_copy(zinit.at[pl.ds(s * 640, 640)],
                    accs.at[pl.ds(s * 640, 640)])
    plsc.subcore_barrier()

    def body(j, carry):
        pltpu.sync_copy(ones_v, accs.at[dstb.at[j]], add=True)
        return carry

    lax.fori_loop(0, 40, body, 0)
    plsc.subcore_barrier()
    pltpu.sync_copy(accs.at[pl.ds(s * 640, 640)],
                    out.at[pl.ds(c * NPAD + s * 640, 640)])


def _sc_degree(dst2d, ones_hbm, zinit):
    k = pl.kernel(
        _deg_body,
        out_type=jax.ShapeDtypeStruct((2 * NPAD, 128), jnp.float32),
        mesh=_MESH,
        scratch_types=[
            pltpu.VMEM((40, 128), jnp.int32),
            pltpu.VMEM((128, 128), jnp.float32),
            pltpu.VMEM_SHARED((NPAD, 128), jnp.float32),
        ],
    )
    return k(dst2d, ones_hbm, zinit)


SS = 40


def _prop_body(colsplit, nch, table, src2d, dst2d, zinit, out,
               srcb, dstb, rows0, rows1, accs, sem0, sem1):
    c = lax.axis_index("c")
    s = lax.axis_index("s")
    if colsplit:
        base = s * nch
    else:
        base = (c * NS + s) * nch

    pltpu.sync_copy(zinit.at[pl.ds(s * 640, 640)], accs.at[pl.ds(s * 640, 640)])
    plsc.subcore_barrier()

    for stage in range(nch // SS):
        sb = base + stage * SS
        pltpu.sync_copy(src2d.at[pl.ds(sb, SS)], srcb)
        pltpu.sync_copy(dst2d.at[pl.ds(sb, SS)], dstb)
        off = c * NPAD

        def addoff(j, carry):
            for k in range(8):
                srcb[j, pl.ds(16 * k, 16)] = srcb[j, pl.ds(16 * k, 16)] + off
            return carry

        lax.fori_loop(0, SS, addoff, 0)

        pltpu.async_copy(table.at[srcb.at[0]], rows0, sem0)
        pltpu.async_copy(table.at[srcb.at[1]], rows1, sem1)

        def pair(i, carry):
            j0 = 2 * i
            pltpu.make_async_copy(table.at[srcb.at[j0]], rows0, sem0).wait()
            pltpu.sync_copy(rows0, accs.at[dstb.at[j0]], add=True)
            pltpu.async_copy(table.at[srcb.at[j0 + 2]], rows0, sem0)
            pltpu.make_async_copy(table.at[srcb.at[j0 + 1]], rows1, sem1).wait()
            pltpu.sync_copy(rows1, accs.at[dstb.at[j0 + 1]], add=True)
            pltpu.async_copy(table.at[srcb.at[j0 + 3]], rows1, sem1)
            return carry

        lax.fori_loop(0, SS // 2 - 1, pair, 0)
        pltpu.make_async_copy(table.at[srcb.at[SS - 2]], rows0, sem0).wait()
        pltpu.sync_copy(rows0, accs.at[dstb.at[SS - 2]], add=True)
        pltpu.make_async_copy(table.at[srcb.at[SS - 1]], rows1, sem1).wait()
        pltpu.sync_copy(rows1, accs.at[dstb.at[SS - 1]], add=True)

    plsc.subcore_barrier()
    pltpu.sync_copy(accs.at[pl.ds(s * 640, 640)],
                    out.at[pl.ds(c * NPAD + s * 640, 640)])


def _sc_propagate(table, src2d, dst2d, zinit, colsplit):
    nch = (EPAD // 128) // NS if colsplit else (EPAD // 128) // (NC * NS)
    k = pl.kernel(
        functools.partial(_prop_body, colsplit, nch),
        out_type=jax.ShapeDtypeStruct((2 * NPAD, 128), jnp.float32),
        mesh=_MESH,
        scratch_types=[
            pltpu.VMEM((SS, 128), jnp.int32),
            pltpu.VMEM((SS, 128), jnp.int32),
            pltpu.VMEM((128, 128), jnp.float32),
            pltpu.VMEM((128, 128), jnp.float32),
            pltpu.VMEM_SHARED((NPAD, 128), jnp.float32),
            pltpu.SemaphoreType.DMA,
            pltpu.SemaphoreType.DMA,
        ],
    )
    return k(table, src2d, dst2d, zinit)


def _encode1_body(x_ref, w_ref, deg_ref, p_ref, dinv_ref):
    d = deg_ref[0] + deg_ref[1] + 1.0
    dcol = lax.rsqrt(d[:, 0:1])
    xw = jnp.dot(x_ref[...], w_ref[...], preferred_element_type=jnp.float32)
    p = dcol * xw
    p_ref[0] = p[:, :128]
    p_ref[1] = p[:, 128:]
    dinv_ref[...] = jnp.broadcast_to(dcol, (BT, 128))


def _tc_encode1(x, W1, deg2):
    return pl.pallas_call(
        _encode1_body,
        grid=(NPAD // BT,),
        in_specs=[
            pl.BlockSpec((BT, F_IN), lambda i: (i, 0)),
            pl.BlockSpec((F_IN, HID), lambda i: (0, 0)),
            pl.BlockSpec((2, BT, 128), lambda i: (0, i, 0)),
        ],
        out_specs=[
            pl.BlockSpec((2, BT, 128), lambda i: (0, i, 0)),
            pl.BlockSpec((BT, 128), lambda i: (i, 0)),
        ],
        out_shape=[
            jax.ShapeDtypeStruct((2, NPAD, 128), jnp.float32),
            jax.ShapeDtypeStruct((NPAD, 128), jnp.float32),
        ],
    )(x, W1, deg2)


def _encode2_body(s_ref, p_ref, dinv_ref, w_ref, q_ref):
    din = dinv_ref[...]
    h0 = jnp.maximum(din * (s_ref[0] + p_ref[0]), 0.0)
    h1 = jnp.maximum(din * (s_ref[1] + p_ref[1]), 0.0)
    h = jnp.concatenate([h0, h1], axis=1)
    q = din * jnp.dot(h, w_ref[...], preferred_element_type=jnp.float32)
    q_ref[0] = q
    q_ref[1] = q


def _tc_encode2(S1, P, dinvb, W2):
    return pl.pallas_call(
        _encode2_body,
        grid=(NPAD // BT,),
        in_specs=[
            pl.BlockSpec((2, BT, 128), lambda i: (0, i, 0)),
            pl.BlockSpec((2, BT, 128), lambda i: (0, i, 0)),
            pl.BlockSpec((BT, 128), lambda i: (i, 0)),
            pl.BlockSpec((HID, CODE), lambda i: (0, 0)),
        ],
        out_specs=pl.BlockSpec((2, BT, 128), lambda i: (0, i, 0)),
        out_shape=jax.ShapeDtypeStruct((2, NPAD, 128), jnp.float32),
    )(S1, P, dinvb, W2)


def _z_body(s_ref, q_ref, dinv_ref, z_ref):
    z_ref[...] = dinv_ref[...] * (s_ref[0] + s_ref[1] + q_ref[0])


def _tc_z(S2, Q2, dinvb):
    return pl.pallas_call(
        _z_body,
        grid=(NPAD // BT,),
        in_specs=[
            pl.BlockSpec((2, BT, 128), lambda i: (0, i, 0)),
            pl.BlockSpec((2, BT, 128), lambda i: (0, i, 0)),
            pl.BlockSpec((BT, 128), lambda i: (i, 0)),
        ],
        out_specs=pl.BlockSpec((BT, 128), lambda i: (i, 0)),
        out_shape=jax.ShapeDtypeStruct((NPAD, 128), jnp.float32),
    )(S2, Q2, dinvb)


def _decode_body(zi_ref, zj_ref, o_ref):
    zz = lax.dot_general(zi_ref[...], zj_ref[...],
                         (((1,), (1,)), ((), ())),
                         preferred_element_type=jnp.float32)
    o_ref[...] = jax.nn.sigmoid(zz)


def _tc_decode(z):
    return pl.pallas_call(
        _decode_body,
        grid=(pl.cdiv(N, BD), pl.cdiv(N, BD)),
        in_specs=[
            pl.BlockSpec((BD, CODE), lambda i, j: (i, 0)),
            pl.BlockSpec((BD, CODE), lambda i, j: (j, 0)),
        ],
        out_specs=pl.BlockSpec((BD, BD), lambda i, j: (i, j)),
        out_shape=jax.ShapeDtypeStruct((N, N), jnp.float32),
    )(z, z)


def kernel(x, edge_index, W1, W2):
    src = edge_index[0].astype(jnp.int32)
    dst = edge_index[1].astype(jnp.int32)
    src2d = jnp.concatenate(
        [src, jnp.zeros((EPAD - E,), jnp.int32)]).reshape(EPAD // 128, 128)
    pad_dst = N + jnp.arange(EPAD - E, dtype=jnp.int32) % (NPAD - N)
    dst2d = jnp.concatenate([dst, pad_dst]).reshape(EPAD // 128, 128)
    zinit = jnp.zeros((NPAD, 128), jnp.float32)
    ones128 = jnp.ones((128, 128), jnp.float32)
    xp = jnp.pad(x, ((0, NPAD - N), (0, 0)))

    deg2 = _sc_degree(dst2d, ones128, zinit).reshape(2, NPAD, 128)
    P, dinvb = _tc_encode1(xp, W1, deg2)
    S1 = _sc_propagate(P.reshape(2 * NPAD, 128), src2d, dst2d, zinit,
                       colsplit=True).reshape(2, NPAD, 128)
    Q2 = _tc_encode2(S1, P, dinvb, W2)
    S2 = _sc_propagate(Q2.reshape(2 * NPAD, 128), src2d, dst2d, zinit,
                       colsplit=False).reshape(2, NPAD, 128)
    z = _tc_z(S2, Q2, dinvb)
    return _tc_decode(z)


# speedup vs baseline: 1.4608x; 1.0029x over previous
import functools

import jax
import jax.numpy as jnp
from jax import lax
from jax.experimental import pallas as pl
from jax.experimental.pallas import tpu as pltpu
from jax.experimental.pallas import tpu_sc as plsc

N = 10000
NPAD = 10240
E = 160000
EPAD = 163840
F_IN = 256
HID = 256
CODE = 128
NC = 2
NS = 16
BT = 512
BD = 2560

_MESH = plsc.VectorSubcoreMesh(
    core_axis_name="c", subcore_axis_name="s", num_cores=NC, num_subcores=NS)


def _deg_body(dst2d, ones_hbm, zinit, out, dstb, ones_v, accs):
    c = lax.axis_index("c")
    s = lax.axis_index("s")
    base = (c * NS + s) * 40
    pltpu.sync_copy(dst2d.at[pl.ds(base, 40)], dstb)
    pltpu.sync_copy(ones_hbm, ones_v)
    pltpu.sync_copy(zinit.at[pl.ds(s * 640, 640)],
                    accs.at[pl.ds(s * 640, 640)])
    plsc.subcore_barrier()

    def body(j, carry):
        pltpu.sync_copy(ones_v, accs.at[dstb.at[j]], add=True)
        return carry

    lax.fori_loop(0, 40, body, 0)
    plsc.subcore_barrier()
    pltpu.sync_copy(accs.at[pl.ds(s * 640, 640)],
                    out.at[pl.ds(c * NPAD + s * 640, 640)])


def _sc_degree(dst2d, ones_hbm, zinit):
    k = pl.kernel(
        _deg_body,
        out_type=jax.ShapeDtypeStruct((2 * NPAD, 128), jnp.float32),
        mesh=_MESH,
        scratch_types=[
            pltpu.VMEM((40, 128), jnp.int32),
            pltpu.VMEM((128, 128), jnp.float32),
            pltpu.VMEM_SHARED((NPAD, 128), jnp.float32),
        ],
    )
    return k(dst2d, ones_hbm, zinit)


SS = 40


def _prop_body(colsplit, nch, table, src2d, dst2d, zinit, out,
               srcb, dstb, rows0, rows1, accs, sem0, sem1):
    c = lax.axis_index("c")
    s = lax.axis_index("s")
    if colsplit:
        base = s * nch
    else:
        base = (c * NS + s) * nch

    pltpu.sync_copy(zinit.at[pl.ds(s * 640, 640)], accs.at[pl.ds(s * 640, 640)])
    plsc.subcore_barrier()

    for stage in range(nch // SS):
        sb = base + stage * SS
        pltpu.sync_copy(src2d.at[pl.ds(sb, SS)], srcb)
        pltpu.sync_copy(dst2d.at[pl.ds(sb, SS)], dstb)
        off = c * NPAD

        def addoff(j, carry):
            for k in range(8):
                srcb[j, pl.ds(16 * k, 16)] = srcb[j, pl.ds(16 * k, 16)] + off
            return carry

        lax.fori_loop(0, SS, addoff, 0)

        pltpu.async_copy(table.at[srcb.at[0]], rows0, sem0)
        pltpu.async_copy(table.at[srcb.at[1]], rows1, sem1)

        def pair(i, carry):
            j0 = 2 * i
            pltpu.make_async_copy(table.at[srcb.at[j0]], rows0, sem0).wait()
            pltpu.sync_copy(rows0, accs.at[dstb.at[j0]], add=True)
            pltpu.async_copy(table.at[srcb.at[j0 + 2]], rows0, sem0)
            pltpu.make_async_copy(table.at[srcb.at[j0 + 1]], rows1, sem1).wait()
            pltpu.sync_copy(rows1, accs.at[dstb.at[j0 + 1]], add=True)
            pltpu.async_copy(table.at[srcb.at[j0 + 3]], rows1, sem1)
            return carry

        lax.fori_loop(0, SS // 2 - 1, pair, 0)
        pltpu.make_async_copy(table.at[srcb.at[SS - 2]], rows0, sem0).wait()
        pltpu.sync_copy(rows0, accs.at[dstb.at[SS - 2]], add=True)
        pltpu.make_async_copy(table.at[srcb.at[SS - 1]], rows1, sem1).wait()
        pltpu.sync_copy(rows1, accs.at[dstb.at[SS - 1]], add=True)

    plsc.subcore_barrier()
    pltpu.sync_copy(accs.at[pl.ds(s * 640, 640)],
                    out.at[pl.ds(c * NPAD + s * 640, 640)])


def _sc_propagate(table, src2d, dst2d, zinit, colsplit):
    nch = (EPAD // 128) // NS if colsplit else (EPAD // 128) // (NC * NS)
    k = pl.kernel(
        functools.partial(_prop_body, colsplit, nch),
        out_type=jax.ShapeDtypeStruct((2 * NPAD, 128), jnp.float32),
        mesh=_MESH,
        scratch_types=[
            pltpu.VMEM((SS, 128), jnp.int32),
            pltpu.VMEM((SS, 128), jnp.int32),
            pltpu.VMEM((128, 128), jnp.float32),
            pltpu.VMEM((128, 128), jnp.float32),
            pltpu.VMEM_SHARED((NPAD, 128), jnp.float32),
            pltpu.SemaphoreType.DMA,
            pltpu.SemaphoreType.DMA,
        ],
    )
    return k(table, src2d, dst2d, zinit)


def _encode1_body(x_ref, w_ref, deg_ref, p_ref, dinv_ref):
    d = deg_ref[0] + deg_ref[1] + 1.0
    dcol = lax.rsqrt(d[:, 0:1])
    xw = jnp.dot(x_ref[...], w_ref[...], preferred_element_type=jnp.float32)
    p = dcol * xw
    p_ref[0] = p[:, :128]
    p_ref[1] = p[:, 128:]
    dinv_ref[...] = jnp.broadcast_to(dcol, (BT, 128))


def _tc_encode1(x, W1, deg2):
    return pl.pallas_call(
        _encode1_body,
        grid=(NPAD // BT,),
        in_specs=[
            pl.BlockSpec((BT, F_IN), lambda i: (i, 0)),
            pl.BlockSpec((F_IN, HID), lambda i: (0, 0)),
            pl.BlockSpec((2, BT, 128), lambda i: (0, i, 0)),
        ],
        out_specs=[
            pl.BlockSpec((2, BT, 128), lambda i: (0, i, 0)),
            pl.BlockSpec((BT, 128), lambda i: (i, 0)),
        ],
        out_shape=[
            jax.ShapeDtypeStruct((2, NPAD, 128), jnp.float32),
            jax.ShapeDtypeStruct((NPAD, 128), jnp.float32),
        ],
    )(x, W1, deg2)


def _encode2_body(s_ref, p_ref, dinv_ref, w_ref, q_ref):
    din = dinv_ref[...]
    h0 = jnp.maximum(din * (s_ref[0] + p_ref[0]), 0.0)
    h1 = jnp.maximum(din * (s_ref[1] + p_ref[1]), 0.0)
    h = jnp.concatenate([h0, h1], axis=1)
    q = din * jnp.dot(h, w_ref[...], preferred_element_type=jnp.float32)
    q_ref[0] = q
    q_ref[1] = q


def _tc_encode2(S1, P, dinvb, W2):
    return pl.pallas_call(
        _encode2_body,
        grid=(NPAD // BT,),
        in_specs=[
            pl.BlockSpec((2, BT, 128), lambda i: (0, i, 0)),
            pl.BlockSpec((2, BT, 128), lambda i: (0, i, 0)),
            pl.BlockSpec((BT, 128), lambda i: (i, 0)),
            pl.BlockSpec((HID, CODE), lambda i: (0, 0)),
        ],
        out_specs=pl.BlockSpec((2, BT, 128), lambda i: (0, i, 0)),
        out_shape=jax.ShapeDtypeStruct((2, NPAD, 128), jnp.float32),
    )(S1, P, dinvb, W2)


def _z_body(s_ref, q_ref, dinv_ref, z_ref):
    z_ref[...] = dinv_ref[...] * (s_ref[0] + s_ref[1] + q_ref[0])


def _tc_z(S2, Q2, dinvb):
    return pl.pallas_call(
        _z_body,
        grid=(NPAD // BT,),
        in_specs=[
            pl.BlockSpec((2, BT, 128), lambda i: (0, i, 0)),
            pl.BlockSpec((2, BT, 128), lambda i: (0, i, 0)),
            pl.BlockSpec((BT, 128), lambda i: (i, 0)),
        ],
        out_specs=pl.BlockSpec((BT, 128), lambda i: (i, 0)),
        out_shape=jax.ShapeDtypeStruct((NPAD, 128), jnp.float32),
    )(S2, Q2, dinvb)


def _decode_body(zi_ref, zj_ref, o_ref):
    zz = lax.dot_general(zi_ref[...], zj_ref[...],
                         (((1,), (1,)), ((), ())),
                         preferred_element_type=jnp.float32)
    o_ref[...] = jax.nn.sigmoid(zz)


def _tc_decode(z):
    return pl.pallas_call(
        _decode_body,
        grid=(pl.cdiv(N, BD), pl.cdiv(N, BD)),
        in_specs=[
            pl.BlockSpec((BD, CODE), lambda i, j: (i, 0)),
            pl.BlockSpec((BD, CODE), lambda i, j: (j, 0)),
        ],
        out_specs=pl.BlockSpec((BD, BD), lambda i, j: (i, j)),
        out_shape=jax.ShapeDtypeStruct((N, N), jnp.float32),
    )(z, z)


def kernel(x, edge_index, W1, W2):
    src = edge_index[0].astype(jnp.int32)
    dst = edge_index[1].astype(jnp.int32)
    src2d = jnp.concatenate(
        [src, jnp.zeros((EPAD - E,), jnp.int32)]).reshape(EPAD // 128, 128)
    pad_dst = N + jnp.arange(EPAD - E, dtype=jnp.int32) % (NPAD - N)
    dst2d = jnp.concatenate([dst, pad_dst]).reshape(EPAD // 128, 128)
    zinit = jnp.zeros((NPAD, 128), jnp.float32)
    ones128 = jnp.ones((128, 128), jnp.float32)
    xp = jnp.pad(x, ((0, NPAD - N), (0, 0)))

    deg2 = _sc_degree(dst2d, ones128, zinit).reshape(2, NPAD, 128)
    P, dinvb = _tc_encode1(xp, W1, deg2)
    S1 = _sc_propagate(P.reshape(2 * NPAD, 128), src2d, dst2d, zinit,
                       colsplit=True).reshape(2, NPAD, 128)
    Q2 = _tc_encode2(S1, P, dinvb, W2)
    S2 = _sc_propagate(Q2.reshape(2 * NPAD, 128), src2d, dst2d, zinit,
                       colsplit=False).reshape(2, NPAD, 128)
    z = _tc_z(S2, Q2, dinvb)
    return _tc_decode(z)
